# baseline (speedup 1.0000x reference)
import jax
import jax.numpy as jnp
from jax import lax
from jax.experimental import pallas as pl
from jax.experimental.pallas import tpu as pltpu
from jax.experimental.pallas import tpu_sc as plsc

D = 32
B = 16384
H = 50
N = B * H
NC, NS = 2, 16
NW = NC * NS
SB = B // NW
PER_W = N // NW
BLK = 10
NBLK = H // BLK


def _format_body(idx_hbm, out_hbm, idx_v):
    wid = lax.axis_index("s") * NC + lax.axis_index("c")
    b0 = wid * SB
    pltpu.sync_copy(idx_hbm.at[:, pl.ds(b0, SB)], idx_v)
    for h in range(H):
        pltpu.sync_copy(idx_v.at[h],
                        out_hbm.at[pl.ds(wid * PER_W + h * SB, SB)])


@jax.jit
def _format_idx(idx_t):
    f = pl.kernel(
        _format_body,
        out_type=jax.ShapeDtypeStruct((N,), jnp.int32),
        mesh=plsc.VectorSubcoreMesh(core_axis_name="c", subcore_axis_name="s"),
        scratch_types=[
            pltpu.VMEM((H, SB), jnp.int32),
        ],
        compiler_params=pltpu.CompilerParams(use_tc_tiling_on_sc=True),
    )
    return f(idx_t)


def _transpose_chunk(rows, tr):
    iota = lax.iota(jnp.int32, 16)

    @plsc.parallel_loop(0, D * (SB // 16), unroll=8)
    def _(i):
        d = i >> 5
        jg = i & 31
        rowv = jg * 16 + iota
        colv = jnp.zeros((16,), jnp.int32) + d
        v = plsc.load_gather(rows, [rowv, colv])
        tr[d, pl.ds(jg * 16, 16)] = v


def _gather_body(idx_hbm, table_hbm, out_hbm, idx_v, rows_v, tr_v,
                 gsem0, gsem1, ssem0, ssem1):
    wid = lax.axis_index("s") * NC + lax.axis_index("c")
    base = wid * PER_W
    b0 = wid * SB
    gsem = (gsem0, gsem1)
    ssem = (ssem0, ssem1)

    pltpu.sync_copy(idx_hbm.at[pl.ds(base, PER_W)], idx_v)

    def block(g, carry):
        stores = [None, None]
        gathers = [None, None]

        def emit(j, c):
            p = j % 2
            gathers[p].wait()
            _transpose_chunk(rows_v.at[p], tr_v.at[p])
            stores[p] = pltpu.make_async_copy(
                tr_v.at[p], out_hbm.at[c, :, pl.ds(b0, SB)], ssem[p])
            stores[p].start()

        for j in range(BLK):
            c = g * BLK + j
            s = j % 2
            if stores[s] is not None:
                stores[s].wait()
            gathers[s] = pltpu.make_async_copy(
                table_hbm.at[idx_v.at[pl.ds(c * SB, SB)]], rows_v.at[s],
                gsem[s])
            gathers[s].start()
            if j > 0:
                emit(j - 1, c - 1)
        emit(BLK - 1, g * BLK + BLK - 1)
        stores[0].wait()
        stores[1].wait()
        return carry

    lax.fori_loop(0, NBLK, block, 0)


@jax.jit
def _gather(idx_flat, table):
    f = pl.kernel(
        _gather_body,
        out_type=jax.ShapeDtypeStruct((H, D, B), jnp.float32),
        mesh=plsc.VectorSubcoreMesh(core_axis_name="c", subcore_axis_name="s"),
        scratch_types=[
            pltpu.VMEM((PER_W,), jnp.int32),
            pltpu.VMEM((2, SB, D), jnp.float32),
            pltpu.VMEM((2, D, SB), jnp.float32),
            pltpu.SemaphoreType.DMA,
            pltpu.SemaphoreType.DMA,
            pltpu.SemaphoreType.DMA,
            pltpu.SemaphoreType.DMA,
        ],
        compiler_params=pltpu.CompilerParams(use_tc_tiling_on_sc=False,
                                             needs_layout_passes=False),
    )
    return f(idx_flat, table)


def kernel(input, embeddings):
    idx_flat = _format_idx(input.T.astype(jnp.int32))
    out = _gather(idx_flat, embeddings)
    return jnp.transpose(out, (2, 0, 1))


# speedup vs baseline: 1.5507x; 1.5507x over previous
import jax
import jax.numpy as jnp
from jax import lax
from jax.experimental import pallas as pl
from jax.experimental.pallas import tpu as pltpu
from jax.experimental.pallas import tpu_sc as plsc

D = 32
B = 16384
H = 50
N = B * H
NC, NS = 2, 16
NW = NC * NS
SB = B // NW
PER_W = N // NW
BLK = 10
NBLK = H // BLK


def _format_body(idx_hbm, out_hbm, idx_v):
    wid = lax.axis_index("s") * NC + lax.axis_index("c")
    b0 = wid * SB
    pltpu.sync_copy(idx_hbm.at[:, pl.ds(b0, SB)], idx_v)
    for h in range(H):
        pltpu.sync_copy(idx_v.at[h],
                        out_hbm.at[pl.ds(wid * PER_W + h * SB, SB)])


@jax.jit
def _format_idx(idx_t):
    f = pl.kernel(
        _format_body,
        out_type=jax.ShapeDtypeStruct((N,), jnp.int32),
        mesh=plsc.VectorSubcoreMesh(core_axis_name="c", subcore_axis_name="s"),
        scratch_types=[
            pltpu.VMEM((H, SB), jnp.int32),
        ],
        compiler_params=pltpu.CompilerParams(use_tc_tiling_on_sc=True),
    )
    return f(idx_t)


def _transpose_chunk(rows, tr):
    iota = lax.iota(jnp.int32, 16)

    @plsc.parallel_loop(0, D * (SB // 16), unroll=8)
    def _(i):
        d0 = i & 31
        rowv = (i >> 5) * 16 + iota
        colv = (d0 + iota) & (D - 1)
        v = plsc.load_gather(rows, [rowv, colv])
        plsc.store_scatter(tr, [colv, rowv], v)


def _gather_body(idx_hbm, table_hbm, out_hbm, idx_v, rows_v, tr_v,
                 gsem0, gsem1, ssem0, ssem1):
    wid = lax.axis_index("s") * NC + lax.axis_index("c")
    base = wid * PER_W
    b0 = wid * SB
    gsem = (gsem0, gsem1)
    ssem = (ssem0, ssem1)

    pltpu.sync_copy(idx_hbm.at[pl.ds(base, PER_W)], idx_v)

    def block(g, carry):
        stores = [None, None]
        gathers = [None, None]

        def emit(j, c):
            p = j % 2
            gathers[p].wait()
            _transpose_chunk(rows_v.at[p], tr_v.at[p])
            stores[p] = pltpu.make_async_copy(
                tr_v.at[p], out_hbm.at[c, :, pl.ds(b0, SB)], ssem[p])
            stores[p].start()

        for j in range(BLK):
            c = g * BLK + j
            s = j % 2
            if stores[s] is not None:
                stores[s].wait()
            gathers[s] = pltpu.make_async_copy(
                table_hbm.at[idx_v.at[pl.ds(c * SB, SB)]], rows_v.at[s],
                gsem[s])
            gathers[s].start()
            if j > 0:
                emit(j - 1, c - 1)
        emit(BLK - 1, g * BLK + BLK - 1)
        stores[0].wait()
        stores[1].wait()
        return carry

    lax.fori_loop(0, NBLK, block, 0)


@jax.jit
def _gather(idx_flat, table):
    f = pl.kernel(
        _gather_body,
        out_type=jax.ShapeDtypeStruct((H, D, B), jnp.float32),
        mesh=plsc.VectorSubcoreMesh(core_axis_name="c", subcore_axis_name="s"),
        scratch_types=[
            pltpu.VMEM((PER_W,), jnp.int32),
            pltpu.VMEM((2, SB, D), jnp.float32),
            pltpu.VMEM((2, D, SB), jnp.float32),
            pltpu.SemaphoreType.DMA,
            pltpu.SemaphoreType.DMA,
            pltpu.SemaphoreType.DMA,
            pltpu.SemaphoreType.DMA,
        ],
        compiler_params=pltpu.CompilerParams(use_tc_tiling_on_sc=False,
                                             needs_layout_passes=False),
    )
    return f(idx_flat, table)


def kernel(input, embeddings):
    idx_flat = _format_idx(input.T.astype(jnp.int32))
    out = _gather(idx_flat, embeddings)
    return jnp.transpose(out, (2, 0, 1))


# speedup vs baseline: 2.9339x; 1.8920x over previous
import jax
import jax.numpy as jnp
from jax import lax
from jax.experimental import pallas as pl
from jax.experimental.pallas import tpu as pltpu
from jax.experimental.pallas import tpu_sc as plsc

D = 32
B = 16384
H = 50
N = B * H
NC, NS = 2, 16
NW = NC * NS
SB = B // NW
PER_W = N // NW
BLK = 10
NBLK = H // BLK


def _format_body(idx_hbm, out_hbm, idx_v):
    wid = lax.axis_index("s") * NC + lax.axis_index("c")
    b0 = wid * SB
    pltpu.sync_copy(idx_hbm.at[:, pl.ds(b0, SB)], idx_v)
    for h in range(H):
        pltpu.sync_copy(idx_v.at[h],
                        out_hbm.at[pl.ds(wid * PER_W + h * SB, SB)])


@jax.jit
def _format_idx(idx_t):
    f = pl.kernel(
        _format_body,
        out_type=jax.ShapeDtypeStruct((N,), jnp.int32),
        mesh=plsc.VectorSubcoreMesh(core_axis_name="c", subcore_axis_name="s"),
        scratch_types=[
            pltpu.VMEM((H, SB), jnp.int32),
        ],
        compiler_params=pltpu.CompilerParams(use_tc_tiling_on_sc=True),
    )
    return f(idx_t)


V = 1000000
TW = 512
NFB = V // TW
TAIL = V - NFB * TW
SLOTS = 62


def _transpose_diag(src, dst, jgs):
    iota = lax.iota(jnp.int32, 16)

    @plsc.parallel_loop(0, D * jgs, unroll=8)
    def _(i):
        d0 = i & (D - 1)
        jv = (i >> 5) * 16 + iota
        dv = (d0 + iota) & (D - 1)
        v = plsc.load_gather(src, [dv, jv])
        plsc.store_scatter(dst, [jv * D + dv], v)


def _tformat_body(embt_hbm, tail_hbm, t2_hbm, src0_v, src1_v, dst0_v, dst1_v,
                  isem0, isem1, osem0, osem1):
    wid = lax.axis_index("s") * NC + lax.axis_index("c")
    src_b = (src0_v, src1_v)
    dst_b = (dst0_v, dst1_v)
    isem = (isem0, isem1)
    osem = (osem0, osem1)

    def blk_id(slot):
        return wid + NW * slot

    def start_in(slot, par):
        @pl.when(blk_id(slot) < NFB)
        def _():
            pltpu.make_async_copy(
                embt_hbm.at[:, pl.ds(blk_id(slot) * TW, TW)], src_b[par],
                isem[par]).start()

    def wait_in(par):
        pltpu.make_async_copy(
            embt_hbm.at[:, pl.ds(0, TW)], src_b[par], isem[par]).wait()

    def wait_out(par):
        pltpu.make_async_copy(
            dst_b[par], t2_hbm.at[pl.ds(0, TW * D)], osem[par]).wait()

    start_in(0, 0)

    def pair(k, carry):
        for par in range(2):
            slot = 2 * k + par
            start_in(slot + 1, 1 - par)

            @pl.when(blk_id(slot) < NFB)
            def _():
                wait_in(par)

                @pl.when(slot >= 2)
                def _():
                    wait_out(par)

                _transpose_diag(src_b[par], dst_b[par], TW // 16)
                pltpu.make_async_copy(
                    dst_b[par],
                    t2_hbm.at[pl.ds(blk_id(slot) * TW * D, TW * D)],
                    osem[par]).start()
        return carry

    lax.fori_loop(0, SLOTS // 2, pair, 0)

    wait_out(0)
    wait_out(1)

    @pl.when(wid == NW - 1)
    def _():
        pltpu.sync_copy(tail_hbm, dst0_v.at[pl.ds(0, TAIL * D)])
        pltpu.sync_copy(dst0_v.at[pl.ds(0, TAIL * D)],
                        t2_hbm.at[pl.ds(NFB * TW * D, TAIL * D)])


@jax.jit
def _format_table(emb_t, tail_flat):
    f = pl.kernel(
        _tformat_body,
        out_type=jax.ShapeDtypeStruct((V * D,), jnp.float32),
        mesh=plsc.VectorSubcoreMesh(core_axis_name="c", subcore_axis_name="s"),
        scratch_types=[
            pltpu.VMEM((D, TW), jnp.float32),
            pltpu.VMEM((D, TW), jnp.float32),
            pltpu.VMEM((TW * D,), jnp.float32),
            pltpu.VMEM((TW * D,), jnp.float32),
            pltpu.SemaphoreType.DMA,
            pltpu.SemaphoreType.DMA,
            pltpu.SemaphoreType.DMA,
            pltpu.SemaphoreType.DMA,
        ],
        compiler_params=pltpu.CompilerParams(use_tc_tiling_on_sc=True,
                                             needs_layout_passes=False),
    )
    return f(emb_t, tail_flat)


def _transpose_chunk(rows, tr):
    iota = lax.iota(jnp.int32, 16)

    @plsc.parallel_loop(0, D * (SB // 16), unroll=8)
    def _(i):
        d0 = i & 31
        rowv = (i >> 5) * 16 + iota
        colv = (d0 + iota) & (D - 1)
        v = plsc.load_gather(rows, [rowv, colv])
        plsc.store_scatter(tr, [colv, rowv], v)


def _gather_body(idx_hbm, table_hbm, out_hbm, idx_v, rows_v, tr_v,
                 gsem0, gsem1, ssem0, ssem1):
    wid = lax.axis_index("s") * NC + lax.axis_index("c")
    base = wid * PER_W
    b0 = wid * SB
    gsem = (gsem0, gsem1)
    ssem = (ssem0, ssem1)

    pltpu.sync_copy(idx_hbm.at[pl.ds(base, PER_W)], idx_v)

    def block(g, carry):
        stores = [None, None]
        gathers = [None, None]

        def emit(j, c):
            p = j % 2
            gathers[p].wait()
            _transpose_chunk(rows_v.at[p], tr_v.at[p])
            stores[p] = pltpu.make_async_copy(
                tr_v.at[p], out_hbm.at[c, :, pl.ds(b0, SB)], ssem[p])
            stores[p].start()

        for j in range(BLK):
            c = g * BLK + j
            s = j % 2
            if stores[s] is not None:
                stores[s].wait()
            gathers[s] = pltpu.make_async_copy(
                table_hbm.at[idx_v.at[pl.ds(c * SB, SB)]], rows_v.at[s],
                gsem[s])
            gathers[s].start()
            if j > 0:
                emit(j - 1, c - 1)
        emit(BLK - 1, g * BLK + BLK - 1)
        stores[0].wait()
        stores[1].wait()
        return carry

    lax.fori_loop(0, NBLK, block, 0)


@jax.jit
def _gather(idx_flat, table):
    f = pl.kernel(
        _gather_body,
        out_type=jax.ShapeDtypeStruct((H, D, B), jnp.float32),
        mesh=plsc.VectorSubcoreMesh(core_axis_name="c", subcore_axis_name="s"),
        scratch_types=[
            pltpu.VMEM((PER_W,), jnp.int32),
            pltpu.VMEM((2, SB, D), jnp.float32),
            pltpu.VMEM((2, D, SB), jnp.float32),
            pltpu.SemaphoreType.DMA,
            pltpu.SemaphoreType.DMA,
            pltpu.SemaphoreType.DMA,
            pltpu.SemaphoreType.DMA,
        ],
        compiler_params=pltpu.CompilerParams(use_tc_tiling_on_sc=False,
                                             needs_layout_passes=False),
    )
    return f(idx_flat, table)


def kernel(input, embeddings):
    idx_flat = _format_idx(input.T.astype(jnp.int32))
    tail_flat = embeddings[NFB * TW:, :].reshape(-1)
    table_rm = _format_table(embeddings.T, tail_flat).reshape(V, D)
    out = _gather(idx_flat, table_rm)
    return jnp.transpose(out, (2, 0, 1))


# speedup vs baseline: 2.9989x; 1.0221x over previous
import jax
import jax.numpy as jnp
from jax import lax
from jax.experimental import pallas as pl
from jax.experimental.pallas import tpu as pltpu
from jax.experimental.pallas import tpu_sc as plsc

D = 32
B = 16384
H = 50
N = B * H
NC, NS = 2, 16
NW = NC * NS
SB = B // NW
PER_W = N // NW
BLK = 10
NBLK = H // BLK


def _format_body(idx_hbm, out_hbm, idx_v):
    wid = lax.axis_index("s") * NC + lax.axis_index("c")
    b0 = wid * SB
    pltpu.sync_copy(idx_hbm.at[:, pl.ds(b0, SB)], idx_v)
    for h in range(H):
        pltpu.sync_copy(idx_v.at[h],
                        out_hbm.at[pl.ds(wid * PER_W + h * SB, SB)])


@jax.jit
def _format_idx(idx_t):
    f = pl.kernel(
        _format_body,
        out_type=jax.ShapeDtypeStruct((N,), jnp.int32),
        mesh=plsc.VectorSubcoreMesh(core_axis_name="c", subcore_axis_name="s"),
        scratch_types=[
            pltpu.VMEM((H, SB), jnp.int32),
        ],
        compiler_params=pltpu.CompilerParams(use_tc_tiling_on_sc=True),
    )
    return f(idx_t)


V = 1000000
TW = 512
NFB = V // TW
TAIL = V - NFB * TW
SLOTS = 62


def _transpose_diag(src, dst, jgs):
    iota = lax.iota(jnp.int32, 16)

    @plsc.parallel_loop(0, D * jgs, unroll=16)
    def _(i):
        d0 = i & (D - 1)
        jv = (i >> 5) * 16 + iota
        dv = (d0 + iota) & (D - 1)
        v = plsc.load_gather(src, [dv, jv])
        plsc.store_scatter(dst, [jv * D + dv], v)


def _tformat_body(embt_hbm, tail_hbm, t2_hbm, src0_v, src1_v, dst0_v, dst1_v,
                  isem0, isem1, osem0, osem1):
    wid = lax.axis_index("s") * NC + lax.axis_index("c")
    src_b = (src0_v, src1_v)
    dst_b = (dst0_v, dst1_v)
    isem = (isem0, isem1)
    osem = (osem0, osem1)

    def blk_id(slot):
        return wid + NW * slot

    def start_in(slot, par):
        @pl.when(blk_id(slot) < NFB)
        def _():
            pltpu.make_async_copy(
                embt_hbm.at[:, pl.ds(blk_id(slot) * TW, TW)], src_b[par],
                isem[par]).start()

    def wait_in(par):
        pltpu.make_async_copy(
            embt_hbm.at[:, pl.ds(0, TW)], src_b[par], isem[par]).wait()

    def wait_out(par):
        pltpu.make_async_copy(
            dst_b[par], t2_hbm.at[pl.ds(0, TW * D)], osem[par]).wait()

    start_in(0, 0)

    def pair(k, carry):
        for par in range(2):
            slot = 2 * k + par
            start_in(slot + 1, 1 - par)

            @pl.when(blk_id(slot) < NFB)
            def _():
                wait_in(par)

                @pl.when(slot >= 2)
                def _():
                    wait_out(par)

                _transpose_diag(src_b[par], dst_b[par], TW // 16)
                pltpu.make_async_copy(
                    dst_b[par],
                    t2_hbm.at[pl.ds(blk_id(slot) * TW * D, TW * D)],
                    osem[par]).start()
        return carry

    lax.fori_loop(0, SLOTS // 2, pair, 0)

    wait_out(0)
    wait_out(1)

    @pl.when(wid == NW - 1)
    def _():
        pltpu.sync_copy(tail_hbm, dst0_v.at[pl.ds(0, TAIL * D)])
        pltpu.sync_copy(dst0_v.at[pl.ds(0, TAIL * D)],
                        t2_hbm.at[pl.ds(NFB * TW * D, TAIL * D)])


@jax.jit
def _format_table(emb_t, tail_flat):
    f = pl.kernel(
        _tformat_body,
        out_type=jax.ShapeDtypeStruct((V * D,), jnp.float32),
        mesh=plsc.VectorSubcoreMesh(core_axis_name="c", subcore_axis_name="s"),
        scratch_types=[
            pltpu.VMEM((D, TW), jnp.float32),
            pltpu.VMEM((D, TW), jnp.float32),
            pltpu.VMEM((TW * D,), jnp.float32),
            pltpu.VMEM((TW * D,), jnp.float32),
            pltpu.SemaphoreType.DMA,
            pltpu.SemaphoreType.DMA,
            pltpu.SemaphoreType.DMA,
            pltpu.SemaphoreType.DMA,
        ],
        compiler_params=pltpu.CompilerParams(use_tc_tiling_on_sc=True,
                                             needs_layout_passes=False),
    )
    return f(emb_t, tail_flat)


def _transpose_chunk(rows, tr):
    iota = lax.iota(jnp.int32, 16)

    @plsc.parallel_loop(0, D * (SB // 16), unroll=16)
    def _(i):
        d0 = i & 31
        rowv = (i >> 5) * 16 + iota
        colv = (d0 + iota) & (D - 1)
        v = plsc.load_gather(rows, [rowv, colv])
        plsc.store_scatter(tr, [colv, rowv], v)


def _gather_body(idx_hbm, table_hbm, out_hbm, idx_v, rows_v, tr_v,
                 gsem0, gsem1, ssem0, ssem1):
    wid = lax.axis_index("s") * NC + lax.axis_index("c")
    base = wid * PER_W
    b0 = wid * SB
    gsem = (gsem0, gsem1)
    ssem = (ssem0, ssem1)

    pltpu.sync_copy(idx_hbm.at[pl.ds(base, PER_W)], idx_v)

    def block(g, carry):
        stores = [None, None]
        gathers = [None, None]

        def emit(j, c):
            p = j % 2
            gathers[p].wait()
            _transpose_chunk(rows_v.at[p], tr_v.at[p])
            stores[p] = pltpu.make_async_copy(
                tr_v.at[p], out_hbm.at[c, :, pl.ds(b0, SB)], ssem[p])
            stores[p].start()

        for j in range(BLK):
            c = g * BLK + j
            s = j % 2
            if stores[s] is not None:
                stores[s].wait()
            gathers[s] = pltpu.make_async_copy(
                table_hbm.at[idx_v.at[pl.ds(c * SB, SB)]], rows_v.at[s],
                gsem[s])
            gathers[s].start()
            if j > 0:
                emit(j - 1, c - 1)
        emit(BLK - 1, g * BLK + BLK - 1)
        stores[0].wait()
        stores[1].wait()
        return carry

    lax.fori_loop(0, NBLK, block, 0)


@jax.jit
def _gather(idx_flat, table):
    f = pl.kernel(
        _gather_body,
        out_type=jax.ShapeDtypeStruct((H, D, B), jnp.float32),
        mesh=plsc.VectorSubcoreMesh(core_axis_name="c", subcore_axis_name="s"),
        scratch_types=[
            pltpu.VMEM((PER_W,), jnp.int32),
            pltpu.VMEM((2, SB, D), jnp.float32),
            pltpu.VMEM((2, D, SB), jnp.float32),
            pltpu.SemaphoreType.DMA,
            pltpu.SemaphoreType.DMA,
            pltpu.SemaphoreType.DMA,
            pltpu.SemaphoreType.DMA,
        ],
        compiler_params=pltpu.CompilerParams(use_tc_tiling_on_sc=False,
                                             needs_layout_passes=False),
    )
    return f(idx_flat, table)


def kernel(input, embeddings):
    idx_flat = _format_idx(input.T.astype(jnp.int32))
    tail_flat = embeddings[NFB * TW:, :].reshape(-1)
    table_rm = _format_table(embeddings.T, tail_flat).reshape(V, D)
    out = _gather(idx_flat, table_rm)
    return jnp.transpose(out, (2, 0, 1))
